# Initial kernel scaffold; baseline (speedup 1.0000x reference)
import jax
import jax.numpy as jnp
from jax.experimental import pallas as pl


def kernel(confounders, treatment, patient, corpus_embeddings, W_pe, b_pe, W_s1, b_s1, W_s2, b_s2, W_s3, b_s3, W_h1, b_h1, W_h2, b_h2):
    raise NotImplementedError("write your pallas kernel here")


# speedup vs baseline: 6.6254x; 6.6254x over previous
import functools

import jax
import jax.numpy as jnp
from jax import lax
from jax.experimental import pallas as pl
from jax.experimental.pallas import tpu as pltpu
from jax.experimental.pallas import tpu_sc as plsc

B = 1024
ED = 128
ND = 100000
K = 16
CHUNK = 2048
NCHUNK = 49
NDP = NCHUNK * CHUNK
BSIZE = 8
NLANE = CHUNK // BSIZE
NBUCKET = NCHUNK * NLANE
NCAND = K * BSIZE

_SC_NC = 2
_SC_NS = 16
_SC_NW = _SC_NC * _SC_NS

_NEG = -3e38


def _normalize_rows(x):
    n = jnp.sqrt(jnp.sum(x * x, axis=-1, keepdims=True))
    return x / jnp.maximum(n, 1e-12)


def _pe_body(patient_ref, wpe_ref, bpe_ref, out_ref):
    x = jnp.dot(patient_ref[...], wpe_ref[...],
                preferred_element_type=jnp.float32) + bpe_ref[...][None, :]
    out_ref[...] = _normalize_rows(x)


def _sim_body(pe_ref, corpus_ref, mask_ref, out_ref):
    chunk = corpus_ref[...]
    cn = _normalize_rows(chunk)
    sim = lax.dot_general(pe_ref[...], cn, (((1,), (1,)), ((), ())),
                          preferred_element_type=jnp.float32)
    sim = sim + mask_ref[0]
    f = sim[:, 0:NLANE]
    for m in range(1, BSIZE):
        f = jnp.maximum(f, sim[:, m * NLANE:(m + 1) * NLANE])
    out_ref[...] = f


def _bucket_topk_body(bm_ref, cand_ref):
    s = bm_ref[...]
    iota = lax.broadcasted_iota(jnp.int32, s.shape, 1)
    off = lax.broadcasted_iota(jnp.int32, (1, BSIZE), 1) * NLANE
    cols = []
    for _ in range(K):
        vmax = jnp.max(s, axis=1, keepdims=True)
        sel = jnp.min(jnp.where(s == vmax, iota, jnp.int32(2**30)),
                      axis=1, keepdims=True)
        s = jnp.where(iota == sel, _NEG, s)
        base = (sel // NLANE) * CHUNK + (sel % NLANE)
        cols.append(base + off)
    cand_ref[...] = jnp.concatenate(cols, axis=1)


def _rescore_body(ce_ref, ci_ref, pe_ref, sc_ref, ix_ref):
    ce = ce_ref[...]
    cn = _normalize_rows(ce)
    cnb = cn.astype(jnp.bfloat16).astype(jnp.float32)
    peb = pe_ref[...].astype(jnp.bfloat16).astype(jnp.float32)
    sim = jnp.sum(cnb * peb[:, None, :], axis=2)
    gid = ci_ref[...]
    scs, ixs = [], []
    s = sim
    for _ in range(K):
        vmax = jnp.max(s, axis=1, keepdims=True)
        selg = jnp.min(jnp.where(s == vmax, gid, jnp.int32(2**30)),
                       axis=1, keepdims=True)
        scs.append(vmax)
        ixs.append(selg)
        s = jnp.where(gid == selg, _NEG, s)
    sc_ref[...] = jnp.concatenate(scs, axis=1)
    ix_ref[...] = jnp.concatenate(ixs, axis=1)


def _mlp_body(conf_ref, retr_ref, tr_ref, ws1_ref, bs1_ref, ws2_ref, bs2_ref,
              ws3_ref, bs3_ref, wh1_ref, bh1_ref, wh2_ref, bh2_ref,
              out_ref, cf_ref, sh_ref):
    h = jnp.dot(conf_ref[...], ws1_ref[0:64, :],
                preferred_element_type=jnp.float32)
    h = h + jnp.dot(retr_ref[...], ws1_ref[64:, :],
                    preferred_element_type=jnp.float32)
    h = jnp.maximum(h + bs1_ref[...][None, :], 0.0)
    h = jnp.maximum(jnp.dot(h, ws2_ref[...], preferred_element_type=jnp.float32)
                    + bs2_ref[...][None, :], 0.0)
    sh = jnp.dot(h, ws3_ref[...], preferred_element_type=jnp.float32) \
        + bs3_ref[...][None, :]
    sh_ref[...] = sh
    cfs = []
    for t in range(2):
        ht = jnp.maximum(jnp.dot(sh, wh1_ref[t],
                                 preferred_element_type=jnp.float32)
                         + bh1_ref[t][None, :], 0.0)
        cf_t = jnp.dot(ht, wh2_ref[t][:, None],
                       preferred_element_type=jnp.float32) + bh2_ref[t]
        cfs.append(cf_t)
    cf = jnp.concatenate(cfs, axis=1)
    cf_ref[...] = cf
    out_ref[...] = jnp.sum(cf * tr_ref[...], axis=1, keepdims=True)


def _sc_gather_candidates(table, idx):
    n = idx.shape[0]
    per_w = n // _SC_NW
    step = 512
    nstep = per_w // step
    mesh = plsc.VectorSubcoreMesh(core_axis_name="c", subcore_axis_name="s")

    @functools.partial(
        pl.kernel, mesh=mesh,
        out_type=jax.ShapeDtypeStruct((n, ED), jnp.float32),
        compiler_params=pltpu.CompilerParams(use_tc_tiling_on_sc=False),
        scratch_types=[
            pltpu.VMEM((nstep, step), jnp.int32),
            pltpu.VMEM((step, ED), jnp.float32),
            pltpu.SemaphoreType.DMA,
        ],
    )
    def k(table_hbm, idx_hbm, out_hbm, idx_v, rows_v, sem):
        wid = lax.axis_index("s") * _SC_NC + lax.axis_index("c")
        base = wid * per_w
        for j in range(nstep):
            pltpu.sync_copy(idx_hbm.at[pl.ds(base + j * step, step)],
                            idx_v.at[j])
            pltpu.async_copy(table_hbm.at[idx_v.at[j]], rows_v, sem).wait()
            pltpu.sync_copy(rows_v, out_hbm.at[pl.ds(base + j * step, step)])

    return k(table, idx)


def _sc_gather_final(table, idx):
    n = idx.shape[0]
    per_w = n // _SC_NW
    mesh = plsc.VectorSubcoreMesh(core_axis_name="c", subcore_axis_name="s")

    @functools.partial(
        pl.kernel, mesh=mesh,
        out_type=jax.ShapeDtypeStruct((n, ED), jnp.float32),
        compiler_params=pltpu.CompilerParams(use_tc_tiling_on_sc=False),
        scratch_types=[
            pltpu.VMEM((per_w,), jnp.int32),
            pltpu.VMEM((per_w, ED), jnp.float32),
            pltpu.SemaphoreType.DMA,
        ],
    )
    def k(table_hbm, idx_hbm, out_hbm, idx_v, rows_v, sem):
        wid = lax.axis_index("s") * _SC_NC + lax.axis_index("c")
        base = wid * per_w
        pltpu.sync_copy(idx_hbm.at[pl.ds(base, per_w)], idx_v)
        pltpu.async_copy(table_hbm.at[idx_v], rows_v, sem).wait()
        pltpu.sync_copy(rows_v, out_hbm.at[pl.ds(base, per_w)])

    return k(table, idx)


def kernel(confounders, treatment, patient, corpus_embeddings,
           W_pe, b_pe, W_s1, b_s1, W_s2, b_s2, W_s3, b_s3,
           W_h1, b_h1, W_h2, b_h2):
    f32 = jnp.float32

    pe_norm = pl.pallas_call(
        _pe_body,
        out_shape=jax.ShapeDtypeStruct((B, ED), f32),
    )(patient, W_pe, b_pe)

    corpus_p = jnp.pad(corpus_embeddings, ((0, NDP - ND), (0, 0)))
    col = jnp.arange(NDP, dtype=jnp.int32)
    maskvec = jnp.where(col < ND, 0.0, _NEG).astype(f32).reshape(NCHUNK, 1, CHUNK)

    bucketmax = pl.pallas_call(
        _sim_body,
        grid=(NCHUNK,),
        in_specs=[
            pl.BlockSpec((B, ED), lambda c: (0, 0)),
            pl.BlockSpec((CHUNK, ED), lambda c: (c, 0)),
            pl.BlockSpec((1, 1, CHUNK), lambda c: (c, 0, 0)),
        ],
        out_specs=pl.BlockSpec((B, NLANE), lambda c: (0, c)),
        out_shape=jax.ShapeDtypeStruct((B, NBUCKET), f32),
    )(pe_norm, corpus_p, maskvec)

    BT2 = 128
    cand = pl.pallas_call(
        _bucket_topk_body,
        grid=(B // BT2,),
        in_specs=[pl.BlockSpec((BT2, NBUCKET), lambda t: (t, 0))],
        out_specs=pl.BlockSpec((BT2, NCAND), lambda t: (t, 0)),
        out_shape=jax.ShapeDtypeStruct((B, NCAND), jnp.int32),
    )(bucketmax)

    cand_emb = _sc_gather_candidates(corpus_p, cand.reshape(-1))
    cand_emb3 = cand_emb.reshape(B, NCAND, ED)

    BT4 = 64
    scores, indices = pl.pallas_call(
        _rescore_body,
        grid=(B // BT4,),
        in_specs=[
            pl.BlockSpec((BT4, NCAND, ED), lambda t: (t, 0, 0)),
            pl.BlockSpec((BT4, NCAND), lambda t: (t, 0)),
            pl.BlockSpec((BT4, ED), lambda t: (t, 0)),
        ],
        out_specs=[
            pl.BlockSpec((BT4, K), lambda t: (t, 0)),
            pl.BlockSpec((BT4, K), lambda t: (t, 0)),
        ],
        out_shape=[
            jax.ShapeDtypeStruct((B, K), f32),
            jax.ShapeDtypeStruct((B, K), jnp.int32),
        ],
    )(cand_emb3, cand, pe_norm)

    retrieved = _sc_gather_final(corpus_p, indices.reshape(-1))
    retr_flat = retrieved.reshape(B, K * ED)

    BT6 = 256
    rep = lambda t: (0, 0)
    rep1 = lambda t: (0,)
    rep3 = lambda t: (0, 0, 0)
    HD = W_s2.shape[0]
    outcome, cf2, shared = pl.pallas_call(
        _mlp_body,
        grid=(B // BT6,),
        in_specs=[
            pl.BlockSpec((BT6, 64), lambda t: (t, 0)),
            pl.BlockSpec((BT6, K * ED), lambda t: (t, 0)),
            pl.BlockSpec((BT6, 2), lambda t: (t, 0)),
            pl.BlockSpec(W_s1.shape, rep),
            pl.BlockSpec((HD,), rep1),
            pl.BlockSpec((HD, HD), rep),
            pl.BlockSpec((HD,), rep1),
            pl.BlockSpec((HD, HD), rep),
            pl.BlockSpec((HD,), rep1),
            pl.BlockSpec((2, HD, HD // 2), rep3),
            pl.BlockSpec((2, HD // 2), rep),
            pl.BlockSpec((2, HD // 2), rep),
            pl.BlockSpec(memory_space=pltpu.SMEM),
        ],
        out_specs=[
            pl.BlockSpec((BT6, 1), lambda t: (t, 0)),
            pl.BlockSpec((BT6, 2), lambda t: (t, 0)),
            pl.BlockSpec((BT6, HD), lambda t: (t, 0)),
        ],
        out_shape=[
            jax.ShapeDtypeStruct((B, 1), f32),
            jax.ShapeDtypeStruct((B, 2), f32),
            jax.ShapeDtypeStruct((B, HD), f32),
        ],
    )(confounders, retr_flat, treatment, W_s1, b_s1, W_s2, b_s2, W_s3, b_s3,
      W_h1, b_h1, W_h2[:, :, 0], b_h2[:, 0])

    return (outcome, scores, indices, cf2.reshape(B, 2, 1), shared)


# speedup vs baseline: 9.1244x; 1.3772x over previous
import functools

import jax
import jax.numpy as jnp
from jax import lax
from jax.experimental import pallas as pl
from jax.experimental.pallas import tpu as pltpu
from jax.experimental.pallas import tpu_sc as plsc

B = 1024
ED = 128
ND = 100000
K = 16
CHUNK = 2048
NCHUNK = 49
BSIZE = 8
NLANE = CHUNK // BSIZE
NG2 = 2 * NCHUNK
NCAND = K * BSIZE

_SC_NC = 2
_SC_NW = 32

_NEG = -3e38


def _normalize_rows(x):
    n = jnp.sqrt(jnp.sum(x * x, axis=-1, keepdims=True))
    return x / jnp.maximum(n, 1e-12)


def _pe_body(patient_ref, wpe_ref, bpe_ref, out_ref):
    x = jnp.dot(patient_ref[...], wpe_ref[...],
                preferred_element_type=jnp.float32) + bpe_ref[...][None, :]
    out_ref[...] = _normalize_rows(x)


def _sim_body(pe_ref, corpus_ref, mask_ref, bm_ref, f2_ref):
    chunk = corpus_ref[...]
    cn = _normalize_rows(chunk)
    sim = lax.dot_general(pe_ref[...], cn, (((1,), (1,)), ((), ())),
                          preferred_element_type=jnp.float32)
    sim = jnp.where(mask_ref[0] > 0.0, sim, _NEG)
    f = sim[:, 0:NLANE]
    for m in range(1, BSIZE):
        f = jnp.maximum(f, sim[:, m * NLANE:(m + 1) * NLANE])
    lo = f[:, 0:128]
    hi = f[:, 128:256]
    bm_ref[0] = lo
    bm_ref[1] = hi
    f2_ref[0] = jnp.concatenate(
        [jnp.max(lo, axis=1)[None, :],
         jnp.max(hi, axis=1)[None, :]], axis=0)


def _l2_topk_body(f2_ref, selg2_ref, selrow_ref):
    s = f2_ref[...]
    g2f = (lax.broadcasted_iota(jnp.int32, s.shape, 0) * 2
           + lax.broadcasted_iota(jnp.int32, s.shape, 1)).astype(jnp.float32)
    riota = lax.broadcasted_iota(jnp.int32, (B, K), 0)
    sels = []
    for _ in range(K):
        vmax = jnp.max(jnp.max(s, axis=1), axis=0)
        vm = vmax[None, None, :]
        t = jnp.where(s == vm, g2f, jnp.float32(3e38))
        self_ = jnp.min(jnp.min(t, axis=1), axis=0)
        s = jnp.where(g2f == self_[None, None, :], _NEG, s)
        sels.append(self_[:, None])
    selg2 = jnp.concatenate(sels, axis=1).astype(jnp.int32)
    selg2_ref[...] = selg2
    selrow_ref[...] = selg2 * B + riota


def _l1_topk_body(cv_ref, selg2_ref, cand_ref):
    cv = cv_ref[...]
    bt = cv.shape[0] // K
    s = cv.reshape(bt, K, 128)
    g2 = selg2_ref[...]
    lanef = lax.broadcasted_iota(jnp.int32, (bt, K, 128), 2).astype(jnp.float32)
    b1f = g2.astype(jnp.float32)[:, :, None] * 128.0 + lanef
    off = lax.broadcasted_iota(jnp.int32, (1, BSIZE), 1) * NLANE
    cols = []
    for _ in range(K):
        vmax = jnp.max(jnp.max(s, axis=1), axis=1, keepdims=True)
        vm = vmax[:, :, None]
        t = jnp.where(s == vm, b1f, jnp.float32(3e38))
        self_ = jnp.min(jnp.min(t, axis=1), axis=1, keepdims=True)
        s = jnp.where(b1f == self_[:, :, None], _NEG, s)
        sel = self_.astype(jnp.int32)
        base = (sel // NLANE) * CHUNK + (sel % NLANE)
        docs = base + off
        cols.append(jnp.minimum(docs, ND - 1))
    cand_ref[...] = jnp.concatenate(cols, axis=1)


def _rescore_body(ce_ref, ci_ref, pe_ref, sc_ref, ix_ref):
    ce = ce_ref[...]
    bt = ce.shape[0] // NCAND
    ones = jnp.full((ED, 1), 1.0, jnp.float32)
    n2 = lax.dot_general(ce * ce, ones, (((1,), (0,)), ((), ())),
                         precision=lax.Precision.HIGHEST,
                         preferred_element_type=jnp.float32)
    n = jnp.sqrt(n2)
    cnb = (ce / jnp.maximum(n, 1e-12)).astype(jnp.bfloat16)
    peb = pe_ref[...].astype(jnp.bfloat16)
    bigT = lax.dot_general(peb, cnb, (((1,), (1,)), ((), ())),
                           preferred_element_type=jnp.float32)
    t3 = bigT.reshape(bt, bt, NCAND)
    r0 = lax.broadcasted_iota(jnp.int32, t3.shape, 0)
    r1 = lax.broadcasted_iota(jnp.int32, t3.shape, 1)
    sim = jnp.sum(jnp.where(r0 == r1, t3, 0.0), axis=0)
    gidf = ci_ref[...].astype(jnp.float32)
    scs, ixs = [], []
    s = sim
    for _ in range(K):
        vmax = jnp.max(s, axis=1, keepdims=True)
        selg = jnp.min(jnp.where(s == vmax, gidf, jnp.float32(3e38)),
                       axis=1, keepdims=True)
        scs.append(vmax)
        ixs.append(selg)
        s = jnp.where(gidf == selg, _NEG, s)
    sc_ref[...] = jnp.concatenate(scs, axis=1)
    ix_ref[...] = jnp.concatenate(ixs, axis=1).astype(jnp.int32)


def _mlp_body(conf_ref, retr_ref, tr_ref, ws1_ref, bs1_ref, ws2_ref, bs2_ref,
              ws3_ref, bs3_ref, wh1_ref, bh1_ref, wh2_ref, bh2_ref,
              out_ref, cf_ref, sh_ref):
    h = jnp.dot(conf_ref[...], ws1_ref[0:64, :],
                preferred_element_type=jnp.float32)
    h = h + jnp.dot(retr_ref[...], ws1_ref[64:, :],
                    preferred_element_type=jnp.float32)
    h = jnp.maximum(h + bs1_ref[...][None, :], 0.0)
    h = jnp.maximum(jnp.dot(h, ws2_ref[...], preferred_element_type=jnp.float32)
                    + bs2_ref[...][None, :], 0.0)
    sh = jnp.dot(h, ws3_ref[...], preferred_element_type=jnp.float32) \
        + bs3_ref[...][None, :]
    sh_ref[...] = sh
    cfs = []
    for t in range(2):
        ht = jnp.maximum(jnp.dot(sh, wh1_ref[t],
                                 preferred_element_type=jnp.float32)
                         + bh1_ref[t][None, :], 0.0)
        cf_t = jnp.dot(ht, wh2_ref[t][:, None],
                       preferred_element_type=jnp.float32) + bh2_ref[t]
        cfs.append(cf_t)
    cf = jnp.concatenate(cfs, axis=1)
    cf_ref[...] = cf
    out_ref[...] = jnp.sum(cf * tr_ref[...], axis=1, keepdims=True)


def _sc_gather_candidates(table, idx):
    n = idx.shape[0]
    per_w = n // _SC_NW
    step = 512
    nstep = per_w // step
    mesh = plsc.VectorSubcoreMesh(core_axis_name="c", subcore_axis_name="s")

    @functools.partial(
        pl.kernel, mesh=mesh,
        out_type=jax.ShapeDtypeStruct((n, ED), jnp.float32),
        compiler_params=pltpu.CompilerParams(use_tc_tiling_on_sc=False),
        scratch_types=[
            pltpu.VMEM((nstep, step), jnp.int32),
            pltpu.VMEM((step, ED), jnp.float32),
            pltpu.SemaphoreType.DMA,
        ],
    )
    def k(table_hbm, idx_hbm, out_hbm, idx_v, rows_v, sem):
        wid = lax.axis_index("s") * _SC_NC + lax.axis_index("c")
        base = wid * per_w
        for j in range(nstep):
            pltpu.sync_copy(idx_hbm.at[pl.ds(base + j * step, step)],
                            idx_v.at[j])
            pltpu.async_copy(table_hbm.at[idx_v.at[j]], rows_v, sem).wait()
            pltpu.sync_copy(rows_v, out_hbm.at[pl.ds(base + j * step, step)])

    return k(table, idx)


def _sc_gather_rows(table, idx):
    n = idx.shape[0]
    d = table.shape[1]
    per_w = n // _SC_NW
    mesh = plsc.VectorSubcoreMesh(core_axis_name="c", subcore_axis_name="s")

    @functools.partial(
        pl.kernel, mesh=mesh,
        out_type=jax.ShapeDtypeStruct((n, d), jnp.float32),
        compiler_params=pltpu.CompilerParams(use_tc_tiling_on_sc=False),
        scratch_types=[
            pltpu.VMEM((per_w,), jnp.int32),
            pltpu.VMEM((per_w, d), jnp.float32),
            pltpu.SemaphoreType.DMA,
        ],
    )
    def k(table_hbm, idx_hbm, out_hbm, idx_v, rows_v, sem):
        wid = lax.axis_index("s") * _SC_NC + lax.axis_index("c")
        base = wid * per_w
        pltpu.sync_copy(idx_hbm.at[pl.ds(base, per_w)], idx_v)
        pltpu.async_copy(table_hbm.at[idx_v], rows_v, sem).wait()
        pltpu.sync_copy(rows_v, out_hbm.at[pl.ds(base, per_w)])

    return k(table, idx)


def kernel(confounders, treatment, patient, corpus_embeddings,
           W_pe, b_pe, W_s1, b_s1, W_s2, b_s2, W_s3, b_s3,
           W_h1, b_h1, W_h2, b_h2):
    f32 = jnp.float32

    pe_norm = pl.pallas_call(
        _pe_body,
        out_shape=jax.ShapeDtypeStruct((B, ED), f32),
    )(patient, W_pe, b_pe)

    col = jnp.arange(NCHUNK * CHUNK, dtype=jnp.int32)
    maskvec = (col < ND).astype(f32).reshape(NCHUNK, 1, CHUNK)

    bmflat, f2 = pl.pallas_call(
        _sim_body,
        grid=(NCHUNK,),
        in_specs=[
            pl.BlockSpec((B, ED), lambda c: (0, 0)),
            pl.BlockSpec((CHUNK, ED), lambda c: (c, 0)),
            pl.BlockSpec((1, 1, CHUNK), lambda c: (c, 0, 0)),
        ],
        out_specs=[
            pl.BlockSpec((2, B, 128), lambda c: (c, 0, 0)),
            pl.BlockSpec((1, 2, B), lambda c: (c, 0, 0)),
        ],
        out_shape=[
            jax.ShapeDtypeStruct((NG2, B, 128), f32),
            jax.ShapeDtypeStruct((NCHUNK, 2, B), f32),
        ],
    )(pe_norm, corpus_embeddings, maskvec)

    selg2, selrow = pl.pallas_call(
        _l2_topk_body,
        out_shape=[
            jax.ShapeDtypeStruct((B, K), jnp.int32),
            jax.ShapeDtypeStruct((B, K), jnp.int32),
        ],
    )(f2)

    childvals = _sc_gather_rows(bmflat.reshape(NG2 * B, 128),
                                selrow.reshape(-1))

    BT2 = 256
    cand = pl.pallas_call(
        _l1_topk_body,
        grid=(B // BT2,),
        in_specs=[
            pl.BlockSpec((BT2 * K, 128), lambda t: (t, 0)),
            pl.BlockSpec((BT2, K), lambda t: (t, 0)),
        ],
        out_specs=pl.BlockSpec((BT2, NCAND), lambda t: (t, 0)),
        out_shape=jax.ShapeDtypeStruct((B, NCAND), jnp.int32),
    )(childvals, selg2)

    cand_emb = _sc_gather_candidates(corpus_embeddings, cand.reshape(-1))

    BT4 = 64
    scores, indices = pl.pallas_call(
        _rescore_body,
        grid=(B // BT4,),
        in_specs=[
            pl.BlockSpec((BT4 * NCAND, ED), lambda t: (t, 0)),
            pl.BlockSpec((BT4, NCAND), lambda t: (t, 0)),
            pl.BlockSpec((BT4, ED), lambda t: (t, 0)),
        ],
        out_specs=[
            pl.BlockSpec((BT4, K), lambda t: (t, 0)),
            pl.BlockSpec((BT4, K), lambda t: (t, 0)),
        ],
        out_shape=[
            jax.ShapeDtypeStruct((B, K), f32),
            jax.ShapeDtypeStruct((B, K), jnp.int32),
        ],
    )(cand_emb, cand, pe_norm)

    retrieved = _sc_gather_rows(corpus_embeddings, indices.reshape(-1))
    retr_flat = retrieved.reshape(B, K * ED)

    BT6 = 256
    rep = lambda t: (0, 0)
    rep1 = lambda t: (0,)
    rep3 = lambda t: (0, 0, 0)
    HD = W_s2.shape[0]
    outcome, cf2, shared = pl.pallas_call(
        _mlp_body,
        grid=(B // BT6,),
        in_specs=[
            pl.BlockSpec((BT6, 64), lambda t: (t, 0)),
            pl.BlockSpec((BT6, K * ED), lambda t: (t, 0)),
            pl.BlockSpec((BT6, 2), lambda t: (t, 0)),
            pl.BlockSpec(W_s1.shape, rep),
            pl.BlockSpec((HD,), rep1),
            pl.BlockSpec((HD, HD), rep),
            pl.BlockSpec((HD,), rep1),
            pl.BlockSpec((HD, HD), rep),
            pl.BlockSpec((HD,), rep1),
            pl.BlockSpec((2, HD, HD // 2), rep3),
            pl.BlockSpec((2, HD // 2), rep),
            pl.BlockSpec((2, HD // 2), rep),
            pl.BlockSpec(memory_space=pltpu.SMEM),
        ],
        out_specs=[
            pl.BlockSpec((BT6, 1), lambda t: (t, 0)),
            pl.BlockSpec((BT6, 2), lambda t: (t, 0)),
            pl.BlockSpec((BT6, HD), lambda t: (t, 0)),
        ],
        out_shape=[
            jax.ShapeDtypeStruct((B, 1), f32),
            jax.ShapeDtypeStruct((B, 2), f32),
            jax.ShapeDtypeStruct((B, HD), f32),
        ],
    )(confounders, retr_flat, treatment, W_s1, b_s1, W_s2, b_s2, W_s3, b_s3,
      W_h1, b_h1, W_h2[:, :, 0], b_h2[:, 0])

    return (outcome, scores, indices, cf2.reshape(B, 2, 1), shared)


# speedup vs baseline: 9.9858x; 1.0944x over previous
import functools

import jax
import jax.numpy as jnp
from jax import lax
from jax.experimental import pallas as pl
from jax.experimental.pallas import tpu as pltpu
from jax.experimental.pallas import tpu_sc as plsc

B = 1024
ED = 128
ND = 100000
K = 16
CHUNK = 2048
NCHUNK = 49
BSIZE = 8
NLANE = CHUNK // BSIZE
NHALF = NLANE // 128
NG2 = NHALF * NCHUNK
NCAND = K * BSIZE

_SC_NC = 2
_SC_NW = 32

_NEG = -3e38


def _normalize_rows(x):
    n = jnp.sqrt(jnp.sum(x * x, axis=-1, keepdims=True))
    return x / jnp.maximum(n, 1e-12)


def _pe_body(patient_ref, wpe_ref, bpe_ref, out_ref):
    x = jnp.dot(patient_ref[...], wpe_ref[...],
                preferred_element_type=jnp.float32) + bpe_ref[...][None, :]
    out_ref[...] = _normalize_rows(x)


def _sim_body(pe_ref, corpus_ref, mask_ref, bm_ref, f2_ref, cnorm_ref):
    c = pl.program_id(0)
    chunk = corpus_ref[...]
    cn = _normalize_rows(chunk)
    cnorm_ref[...] = cn
    sim = lax.dot_general(pe_ref[...], cn, (((1,), (1,)), ((), ())),
                          preferred_element_type=jnp.float32)

    def emit(sim):
        f = sim[:, 0:NLANE]
        for m in range(1, BSIZE):
            f = jnp.maximum(f, sim[:, m * NLANE:(m + 1) * NLANE])
        f2s = []
        for h in range(NHALF):
            part = f[:, h * 128:(h + 1) * 128]
            bm_ref[h] = part
            f2s.append(jnp.max(part, axis=1)[None, :])
        f2_ref[0] = jnp.concatenate(f2s, axis=0)

    @pl.when(c < NCHUNK - 1)
    def _():
        emit(sim)

    @pl.when(c == NCHUNK - 1)
    def _():
        emit(jnp.where(mask_ref[0] > 0.0, sim, _NEG))


def _l2_topk_body(f2_ref, selg2_ref, selrow_ref):
    s = f2_ref[...]
    g2f = (lax.broadcasted_iota(jnp.int32, s.shape, 0) * NHALF
           + lax.broadcasted_iota(jnp.int32, s.shape, 1)).astype(jnp.float32)
    riota = lax.broadcasted_iota(jnp.int32, (B, K), 0)
    sels = []
    for _ in range(K):
        vmax = jnp.max(jnp.max(s, axis=1), axis=0)
        vm = vmax[None, None, :]
        t = jnp.where(s == vm, g2f, jnp.float32(3e38))
        self_ = jnp.min(jnp.min(t, axis=1), axis=0)
        s = jnp.where(g2f == self_[None, None, :], _NEG, s)
        sels.append(self_[:, None])
    selg2 = jnp.concatenate(sels, axis=1).astype(jnp.int32)
    selg2_ref[...] = selg2
    selrow_ref[...] = selg2 * B + riota


def _l1_topk_body(cv_ref, selg2_ref, cand_ref):
    cv = cv_ref[...]
    bt = cv.shape[0] // K
    s = cv.reshape(bt, K, 128)
    g2 = selg2_ref[...]
    lanef = lax.broadcasted_iota(jnp.int32, (bt, K, 128), 2).astype(jnp.float32)
    b1f = g2.astype(jnp.float32)[:, :, None] * 128.0 + lanef
    off = lax.broadcasted_iota(jnp.int32, (1, BSIZE), 1) * NLANE
    cols = []
    for _ in range(K):
        vmax = jnp.max(jnp.max(s, axis=1), axis=1, keepdims=True)
        vm = vmax[:, :, None]
        t = jnp.where(s == vm, b1f, jnp.float32(3e38))
        self_ = jnp.min(jnp.min(t, axis=1), axis=1, keepdims=True)
        s = jnp.where(b1f == self_[:, :, None], _NEG, s)
        sel = self_.astype(jnp.int32)
        base = (sel // NLANE) * CHUNK + (sel % NLANE)
        docs = base + off
        cols.append(jnp.minimum(docs, ND - 1))
    cand_ref[...] = jnp.concatenate(cols, axis=1)


def _rescore_body(ce_ref, ci_ref, pe_ref, sc_ref, ix_ref):
    ce = ce_ref[...]
    bt = ce.shape[0] // NCAND
    bigT = lax.dot_general(pe_ref[...], ce, (((1,), (1,)), ((), ())),
                           preferred_element_type=jnp.float32)
    t3 = bigT.reshape(bt, bt, NCAND)
    r0 = lax.broadcasted_iota(jnp.int32, t3.shape, 0)
    r1 = lax.broadcasted_iota(jnp.int32, t3.shape, 1)
    sim = jnp.sum(jnp.where(r0 == r1, t3, 0.0), axis=0)
    gidf = ci_ref[...].astype(jnp.float32)
    scs, ixs = [], []
    s = sim
    for _ in range(K):
        vmax = jnp.max(s, axis=1, keepdims=True)
        selg = jnp.min(jnp.where(s == vmax, gidf, jnp.float32(3e38)),
                       axis=1, keepdims=True)
        scs.append(vmax)
        ixs.append(selg)
        s = jnp.where(gidf == selg, _NEG, s)
    sc_ref[...] = jnp.concatenate(scs, axis=1)
    ix_ref[...] = jnp.concatenate(ixs, axis=1).astype(jnp.int32)


def _mlp_body(conf_ref, retr_ref, tr_ref, ws1_ref, bs1_ref, ws2_ref, bs2_ref,
              ws3_ref, bs3_ref, wh1_ref, bh1_ref, wh2_ref, bh2_ref,
              out_ref, cf_ref, sh_ref):
    h = jnp.dot(conf_ref[...], ws1_ref[0:64, :],
                preferred_element_type=jnp.float32)
    h = h + jnp.dot(retr_ref[...], ws1_ref[64:, :],
                    preferred_element_type=jnp.float32)
    h = jnp.maximum(h + bs1_ref[...][None, :], 0.0)
    h = jnp.maximum(jnp.dot(h, ws2_ref[...], preferred_element_type=jnp.float32)
                    + bs2_ref[...][None, :], 0.0)
    sh = jnp.dot(h, ws3_ref[...], preferred_element_type=jnp.float32) \
        + bs3_ref[...][None, :]
    sh_ref[...] = sh
    cfs = []
    for t in range(2):
        ht = jnp.maximum(jnp.dot(sh, wh1_ref[t],
                                 preferred_element_type=jnp.float32)
                         + bh1_ref[t][None, :], 0.0)
        cf_t = jnp.dot(ht, wh2_ref[t][:, None],
                       preferred_element_type=jnp.float32) + bh2_ref[t]
        cfs.append(cf_t)
    cf = jnp.concatenate(cfs, axis=1)
    cf_ref[...] = cf
    out_ref[...] = jnp.sum(cf * tr_ref[...], axis=1, keepdims=True)


def _sc_gather_candidates(table, idx):
    n = idx.shape[0]
    per_w = n // _SC_NW
    step = 512
    nstep = per_w // step
    mesh = plsc.VectorSubcoreMesh(core_axis_name="c", subcore_axis_name="s")

    @functools.partial(
        pl.kernel, mesh=mesh,
        out_type=jax.ShapeDtypeStruct((n, ED), jnp.float32),
        compiler_params=pltpu.CompilerParams(use_tc_tiling_on_sc=False),
        scratch_types=[
            pltpu.VMEM((nstep, step), jnp.int32),
            pltpu.VMEM((step, ED), jnp.float32),
            pltpu.SemaphoreType.DMA,
        ],
    )
    def k(table_hbm, idx_hbm, out_hbm, idx_v, rows_v, sem):
        wid = lax.axis_index("s") * _SC_NC + lax.axis_index("c")
        base = wid * per_w
        for j in range(nstep):
            pltpu.sync_copy(idx_hbm.at[pl.ds(base + j * step, step)],
                            idx_v.at[j])
            pltpu.async_copy(table_hbm.at[idx_v.at[j]], rows_v, sem).wait()
            pltpu.sync_copy(rows_v, out_hbm.at[pl.ds(base + j * step, step)])

    return k(table, idx)


def _sc_gather_rows(table, idx):
    n = idx.shape[0]
    d = table.shape[1]
    per_w = n // _SC_NW
    mesh = plsc.VectorSubcoreMesh(core_axis_name="c", subcore_axis_name="s")

    @functools.partial(
        pl.kernel, mesh=mesh,
        out_type=jax.ShapeDtypeStruct((n, d), jnp.float32),
        compiler_params=pltpu.CompilerParams(use_tc_tiling_on_sc=False),
        scratch_types=[
            pltpu.VMEM((per_w,), jnp.int32),
            pltpu.VMEM((per_w, d), jnp.float32),
            pltpu.SemaphoreType.DMA,
        ],
    )
    def k(table_hbm, idx_hbm, out_hbm, idx_v, rows_v, sem):
        wid = lax.axis_index("s") * _SC_NC + lax.axis_index("c")
        base = wid * per_w
        pltpu.sync_copy(idx_hbm.at[pl.ds(base, per_w)], idx_v)
        pltpu.async_copy(table_hbm.at[idx_v], rows_v, sem).wait()
        pltpu.sync_copy(rows_v, out_hbm.at[pl.ds(base, per_w)])

    return k(table, idx)


def kernel(confounders, treatment, patient, corpus_embeddings,
           W_pe, b_pe, W_s1, b_s1, W_s2, b_s2, W_s3, b_s3,
           W_h1, b_h1, W_h2, b_h2):
    f32 = jnp.float32

    pe_norm = pl.pallas_call(
        _pe_body,
        out_shape=jax.ShapeDtypeStruct((B, ED), f32),
    )(patient, W_pe, b_pe)

    col = jnp.arange(NCHUNK * CHUNK, dtype=jnp.int32)
    maskvec = (col < ND).astype(f32).reshape(NCHUNK, 1, CHUNK)

    bmflat, f2, cnorm = pl.pallas_call(
        _sim_body,
        grid=(NCHUNK,),
        in_specs=[
            pl.BlockSpec((B, ED), lambda c: (0, 0)),
            pl.BlockSpec((CHUNK, ED), lambda c: (c, 0)),
            pl.BlockSpec((1, 1, CHUNK), lambda c: (c, 0, 0)),
        ],
        out_specs=[
            pl.BlockSpec((NHALF, B, 128), lambda c: (c, 0, 0)),
            pl.BlockSpec((1, NHALF, B), lambda c: (c, 0, 0)),
            pl.BlockSpec((CHUNK, ED), lambda c: (c, 0)),
        ],
        out_shape=[
            jax.ShapeDtypeStruct((NG2, B, 128), f32),
            jax.ShapeDtypeStruct((NCHUNK, NHALF, B), f32),
            jax.ShapeDtypeStruct((NCHUNK * CHUNK, ED), f32),
        ],
    )(pe_norm, corpus_embeddings, maskvec)

    selg2, selrow = pl.pallas_call(
        _l2_topk_body,
        out_shape=[
            jax.ShapeDtypeStruct((B, K), jnp.int32),
            jax.ShapeDtypeStruct((B, K), jnp.int32),
        ],
    )(f2)

    childvals = _sc_gather_rows(bmflat.reshape(NG2 * B, 128),
                                selrow.reshape(-1))

    BT2 = 256
    cand = pl.pallas_call(
        _l1_topk_body,
        grid=(B // BT2,),
        in_specs=[
            pl.BlockSpec((BT2 * K, 128), lambda t: (t, 0)),
            pl.BlockSpec((BT2, K), lambda t: (t, 0)),
        ],
        out_specs=pl.BlockSpec((BT2, NCAND), lambda t: (t, 0)),
        out_shape=jax.ShapeDtypeStruct((B, NCAND), jnp.int32),
    )(childvals, selg2)

    cand_emb = _sc_gather_candidates(cnorm, cand.reshape(-1))

    BT4 = 64
    scores, indices = pl.pallas_call(
        _rescore_body,
        grid=(B // BT4,),
        in_specs=[
            pl.BlockSpec((BT4 * NCAND, ED), lambda t: (t, 0)),
            pl.BlockSpec((BT4, NCAND), lambda t: (t, 0)),
            pl.BlockSpec((BT4, ED), lambda t: (t, 0)),
        ],
        out_specs=[
            pl.BlockSpec((BT4, K), lambda t: (t, 0)),
            pl.BlockSpec((BT4, K), lambda t: (t, 0)),
        ],
        out_shape=[
            jax.ShapeDtypeStruct((B, K), f32),
            jax.ShapeDtypeStruct((B, K), jnp.int32),
        ],
    )(cand_emb, cand, pe_norm)

    retrieved = _sc_gather_rows(corpus_embeddings, indices.reshape(-1))
    retr_flat = retrieved.reshape(B, K * ED)

    BT6 = 256
    rep = lambda t: (0, 0)
    rep1 = lambda t: (0,)
    rep3 = lambda t: (0, 0, 0)
    HD = W_s2.shape[0]
    outcome, cf2, shared = pl.pallas_call(
        _mlp_body,
        grid=(B // BT6,),
        in_specs=[
            pl.BlockSpec((BT6, 64), lambda t: (t, 0)),
            pl.BlockSpec((BT6, K * ED), lambda t: (t, 0)),
            pl.BlockSpec((BT6, 2), lambda t: (t, 0)),
            pl.BlockSpec(W_s1.shape, rep),
            pl.BlockSpec((HD,), rep1),
            pl.BlockSpec((HD, HD), rep),
            pl.BlockSpec((HD,), rep1),
            pl.BlockSpec((HD, HD), rep),
            pl.BlockSpec((HD,), rep1),
            pl.BlockSpec((2, HD, HD // 2), rep3),
            pl.BlockSpec((2, HD // 2), rep),
            pl.BlockSpec((2, HD // 2), rep),
            pl.BlockSpec(memory_space=pltpu.SMEM),
        ],
        out_specs=[
            pl.BlockSpec((BT6, 1), lambda t: (t, 0)),
            pl.BlockSpec((BT6, 2), lambda t: (t, 0)),
            pl.BlockSpec((BT6, HD), lambda t: (t, 0)),
        ],
        out_shape=[
            jax.ShapeDtypeStruct((B, 1), f32),
            jax.ShapeDtypeStruct((B, 2), f32),
            jax.ShapeDtypeStruct((B, HD), f32),
        ],
    )(confounders, retr_flat, treatment, W_s1, b_s1, W_s2, b_s2, W_s3, b_s3,
      W_h1, b_h1, W_h2[:, :, 0], b_h2[:, 0])

    return (outcome, scores, indices, cf2.reshape(B, 2, 1), shared)
